# Initial kernel scaffold; baseline (speedup 1.0000x reference)
import jax
import jax.numpy as jnp
from jax.experimental import pallas as pl


def kernel(x, W_conv, b_conv, w_fuse1, w_fuse2):
    raise NotImplementedError("write your pallas kernel here")


# speedup vs baseline: 1.8015x; 1.8015x over previous
import functools

import jax
import jax.numpy as jnp
from jax import lax
from jax.experimental import pallas as pl
from jax.experimental.pallas import tpu as pltpu

_C = 96
_T = 4096
_B = 8


def _merge_pass(xb, n, width, fw0, fw1, jc):
    a = xb[:, :n]
    b = xb[:, n:]
    norm = jnp.sqrt(jnp.sum(xb * xb, axis=0, keepdims=True))
    a_n = (a / norm[:, :n]).astype(jnp.bfloat16)
    b_n = (b / norm[:, n:]).astype(jnp.bfloat16)

    ivec = lax.broadcasted_iota(jnp.int32, (n, 1), 0)
    a_row = (ivec // width).astype(jnp.float32)
    a_col = (ivec % width).astype(jnp.float32)

    best_val = jnp.full((n, 1), -jnp.inf, dtype=jnp.float32)
    best_idx = jnp.zeros((n, 1), dtype=jnp.int32)
    for k in range(n // jc):
        off = k * jc
        sim = lax.dot_general(
            a_n, b_n[:, off:off + jc],
            dimension_numbers=(((0,), (0,)), ((), ())),
            preferred_element_type=jnp.float32,
        )
        jvec = lax.broadcasted_iota(jnp.int32, (1, jc), 1) + (n + off)
        b_row = (jvec // width).astype(jnp.float32)
        b_col = (jvec % width).astype(jnp.float32)
        dr = a_row - b_row
        dc = a_col - b_col
        dist = jnp.sqrt(dr * dr + dc * dc)
        spatial = 1.0 / (dist + 1e-6)
        combined = fw0 * sim + fw1 * spatial
        loc_max = jnp.max(combined, axis=1, keepdims=True)
        jj = lax.broadcasted_iota(jnp.int32, (n, jc), 1) + off
        loc_arg = jnp.min(
            jnp.where(combined == loc_max, jj, n), axis=1, keepdims=True
        )
        upd = loc_max > best_val
        best_val = jnp.where(upd, loc_max, best_val)
        best_idx = jnp.where(upd, loc_arg, best_idx)

    outs = []
    for k in range(n // jc):
        off = k * jc
        jj = lax.broadcasted_iota(jnp.int32, (n, jc), 1) + off
        onehot = (best_idx == jj).astype(jnp.float32)
        sums = b[:, off:off + jc] + lax.dot_general(
            a, onehot,
            dimension_numbers=(((1,), (0,)), ((), ())),
            preferred_element_type=jnp.float32,
            precision=lax.Precision.HIGHEST,
        )
        counts = 1.0 + jnp.sum(onehot, axis=0, keepdims=True)
        outs.append(sums / counts)
    return jnp.concatenate(outs, axis=1)


def _kernel_body(x_ref, w_ref, b_ref, fw_ref, out_ref):
    xb = x_ref[0]
    m1 = _merge_pass(xb, _T // 2, 64, fw_ref[0, 0], fw_ref[0, 1], 512)
    m2 = _merge_pass(m1, _T // 4, 45, fw_ref[1, 0], fw_ref[1, 1], 512)
    out = lax.dot_general(
        w_ref[...], m2,
        dimension_numbers=(((1,), (0,)), ((), ())),
        preferred_element_type=jnp.float32,
        precision=lax.Precision.HIGHEST,
    ) + b_ref[...]
    out_ref[0] = out


@jax.jit
def kernel(x, W_conv, b_conv, w_fuse1, w_fuse2):
    B, C, H, W = x.shape
    xc = x.reshape(B, C, H * W)

    def fw(w):
        w = jnp.clip(w, 0.0, 6.0)
        return w / (jnp.sum(w) + 1e-8)

    fws = jnp.stack([fw(w_fuse1), fw(w_fuse2)]).astype(jnp.float32)

    out = pl.pallas_call(
        _kernel_body,
        grid=(B,),
        in_specs=[
            pl.BlockSpec((1, C, H * W), lambda b: (b, 0, 0)),
            pl.BlockSpec((C, C), lambda b: (0, 0)),
            pl.BlockSpec((C, 1), lambda b: (0, 0)),
            pl.BlockSpec(memory_space=pltpu.SMEM),
        ],
        out_specs=pl.BlockSpec((1, C, (H * W) // 4), lambda b: (b, 0, 0)),
        out_shape=jax.ShapeDtypeStruct((B, C, (H * W) // 4), jnp.float32),
        compiler_params=pltpu.CompilerParams(
            dimension_semantics=("parallel",),
        ),
    )(xc, W_conv, b_conv.reshape(C, 1), fws)
    return out.reshape(B, C, H // 2, W // 2)


# speedup vs baseline: 2.2427x; 1.2450x over previous
import functools

import jax
import jax.numpy as jnp
from jax import lax
from jax.experimental import pallas as pl
from jax.experimental.pallas import tpu as pltpu

_C = 96
_T = 4096
_B = 8


def _merge_pass(xb, n, width, fw0, fw1, jc):
    a = xb[:, :n]
    b = xb[:, n:]
    norm = jnp.sqrt(jnp.sum(xb * xb, axis=0, keepdims=True))
    a_n = (a / norm[:, :n]).astype(jnp.bfloat16)
    b_n = (b / norm[:, n:]).astype(jnp.bfloat16)

    ivec = lax.broadcasted_iota(jnp.int32, (n, 1), 0)
    a_row = (ivec // width).astype(jnp.float32)
    a_col = (ivec % width).astype(jnp.float32)

    best_val = jnp.full((n, 1), -jnp.inf, dtype=jnp.float32)
    best_idx = jnp.zeros((n, 1), dtype=jnp.int32)
    for k in range(n // jc):
        off = k * jc
        sim = lax.dot_general(
            a_n, b_n[:, off:off + jc],
            dimension_numbers=(((0,), (0,)), ((), ())),
            preferred_element_type=jnp.float32,
        )
        jvec = lax.broadcasted_iota(jnp.int32, (1, jc), 1) + (n + off)
        b_row = (jvec // width).astype(jnp.float32)
        b_col = (jvec % width).astype(jnp.float32)
        dr = a_row - b_row
        dc = a_col - b_col
        dist = jnp.sqrt(dr * dr + dc * dc)
        spatial = 1.0 / (dist + 1e-6)
        combined = fw0 * sim + fw1 * spatial
        loc_max = jnp.max(combined, axis=1, keepdims=True)
        jj = lax.broadcasted_iota(jnp.int32, (n, jc), 1) + off
        loc_arg = jnp.min(
            jnp.where(combined == loc_max, jj, n), axis=1, keepdims=True
        )
        upd = loc_max > best_val
        best_val = jnp.where(upd, loc_max, best_val)
        best_idx = jnp.where(upd, loc_arg, best_idx)

    a_hi = a.astype(jnp.bfloat16)
    a_lo = (a - a_hi.astype(jnp.float32)).astype(jnp.bfloat16)
    outs = []
    for k in range(n // jc):
        off = k * jc
        jj = lax.broadcasted_iota(jnp.int32, (n, jc), 1) + off
        onehot = (best_idx == jj).astype(jnp.bfloat16)
        dn = (((1,), (0,)), ((), ()))
        sums = (
            b[:, off:off + jc]
            + lax.dot_general(a_hi, onehot, dimension_numbers=dn,
                              preferred_element_type=jnp.float32)
            + lax.dot_general(a_lo, onehot, dimension_numbers=dn,
                              preferred_element_type=jnp.float32)
        )
        counts = 1.0 + jnp.sum(onehot.astype(jnp.float32), axis=0,
                               keepdims=True)
        outs.append(sums / counts)
    return jnp.concatenate(outs, axis=1)


def _kernel_body(x_ref, w_ref, b_ref, fw_ref, out_ref):
    xb = x_ref[0]
    m1 = _merge_pass(xb, _T // 2, 64, fw_ref[0, 0], fw_ref[0, 1], 512)
    m2 = _merge_pass(m1, _T // 4, 45, fw_ref[1, 0], fw_ref[1, 1], 512)
    out = lax.dot_general(
        w_ref[...], m2,
        dimension_numbers=(((1,), (0,)), ((), ())),
        preferred_element_type=jnp.float32,
        precision=lax.Precision.HIGHEST,
    ) + b_ref[...]
    out_ref[0] = out


@jax.jit
def kernel(x, W_conv, b_conv, w_fuse1, w_fuse2):
    B, C, H, W = x.shape
    xc = x.reshape(B, C, H * W)

    def fw(w):
        w = jnp.clip(w, 0.0, 6.0)
        return w / (jnp.sum(w) + 1e-8)

    fws = jnp.stack([fw(w_fuse1), fw(w_fuse2)]).astype(jnp.float32)

    out = pl.pallas_call(
        _kernel_body,
        grid=(B,),
        in_specs=[
            pl.BlockSpec((1, C, H * W), lambda b: (b, 0, 0)),
            pl.BlockSpec((C, C), lambda b: (0, 0)),
            pl.BlockSpec((C, 1), lambda b: (0, 0)),
            pl.BlockSpec(memory_space=pltpu.SMEM),
        ],
        out_specs=pl.BlockSpec((1, C, (H * W) // 4), lambda b: (b, 0, 0)),
        out_shape=jax.ShapeDtypeStruct((B, C, (H * W) // 4), jnp.float32),
        compiler_params=pltpu.CompilerParams(
            dimension_semantics=("parallel",),
        ),
    )(xc, W_conv, b_conv.reshape(C, 1), fws)
    return out.reshape(B, C, H // 2, W // 2)


# speedup vs baseline: 3.2730x; 1.4594x over previous
import jax
import jax.numpy as jnp
from jax import lax
from jax.experimental import pallas as pl
from jax.experimental.pallas import tpu as pltpu

_C = 96
_T = 4096
_B = 8
_JC = 512


def _fill_spatial(tab_ref, n, width, fw1, jc):
    ivec = lax.broadcasted_iota(jnp.int32, (n, 1), 0)
    a_row = (ivec // width).astype(jnp.float32)
    a_col = (ivec % width).astype(jnp.float32)
    for k in range(n // jc):
        off = k * jc
        jvec = lax.broadcasted_iota(jnp.int32, (1, jc), 1) + (n + off)
        b_row = (jvec // width).astype(jnp.float32)
        b_col = (jvec % width).astype(jnp.float32)
        dr = a_row - b_row
        dc = a_col - b_col
        dist = jnp.sqrt(dr * dr + dc * dc)
        spatial = 1.0 / (dist + 1e-6)
        tab_ref[:, off:off + jc] = fw1 * spatial


def _merge_pass(xb, n, fw0, jc, tab_ref):
    a = xb[:, :n]
    b = xb[:, n:]
    norm = jnp.sqrt(jnp.sum(xb * xb, axis=0, keepdims=True))
    a_n = (a / norm[:, :n]).astype(jnp.bfloat16)
    b_n = (b / norm[:, n:]).astype(jnp.bfloat16)

    best_val = jnp.full((n, 1), -jnp.inf, dtype=jnp.float32)
    best_idx = jnp.zeros((n, 1), dtype=jnp.int32)
    for k in range(n // jc):
        off = k * jc
        sim = lax.dot_general(
            a_n, b_n[:, off:off + jc],
            dimension_numbers=(((0,), (0,)), ((), ())),
            preferred_element_type=jnp.float32,
        )
        combined = fw0 * sim + tab_ref[:, off:off + jc]
        loc_max = jnp.max(combined, axis=1, keepdims=True)
        jj = lax.broadcasted_iota(jnp.int32, (n, jc), 1) + off
        loc_arg = jnp.min(
            jnp.where(combined == loc_max, jj, n), axis=1, keepdims=True
        )
        upd = loc_max > best_val
        best_val = jnp.where(upd, loc_max, best_val)
        best_idx = jnp.where(upd, loc_arg, best_idx)

    a_hi = a.astype(jnp.bfloat16)
    a_lo = (a - a_hi.astype(jnp.float32)).astype(jnp.bfloat16)
    outs = []
    for k in range(n // jc):
        off = k * jc
        jj = lax.broadcasted_iota(jnp.int32, (n, jc), 1) + off
        onehot = (best_idx == jj).astype(jnp.bfloat16)
        dn = (((1,), (0,)), ((), ()))
        sums = (
            b[:, off:off + jc]
            + lax.dot_general(a_hi, onehot, dimension_numbers=dn,
                              preferred_element_type=jnp.float32)
            + lax.dot_general(a_lo, onehot, dimension_numbers=dn,
                              preferred_element_type=jnp.float32)
        )
        counts = 1.0 + jnp.sum(onehot.astype(jnp.float32), axis=0,
                               keepdims=True)
        outs.append(sums / counts)
    return jnp.concatenate(outs, axis=1)


def _kernel_body(x_ref, w_ref, b_ref, fw_ref, out_ref, tab1_ref, tab2_ref):
    @pl.when(pl.program_id(0) == 0)
    def _():
        _fill_spatial(tab1_ref, _T // 2, 64, fw_ref[0, 1], _JC)
        _fill_spatial(tab2_ref, _T // 4, 45, fw_ref[1, 1], _JC)

    xb = x_ref[0]
    m1 = _merge_pass(xb, _T // 2, fw_ref[0, 0], _JC, tab1_ref)
    m2 = _merge_pass(m1, _T // 4, fw_ref[1, 0], _JC, tab2_ref)
    out = lax.dot_general(
        w_ref[...], m2,
        dimension_numbers=(((1,), (0,)), ((), ())),
        preferred_element_type=jnp.float32,
        precision=lax.Precision.HIGHEST,
    ) + b_ref[...]
    out_ref[0] = out


@jax.jit
def kernel(x, W_conv, b_conv, w_fuse1, w_fuse2):
    B, C, H, W = x.shape
    xc = x.reshape(B, C, H * W)

    def fw(w):
        w = jnp.clip(w, 0.0, 6.0)
        return w / (jnp.sum(w) + 1e-8)

    fws = jnp.stack([fw(w_fuse1), fw(w_fuse2)]).astype(jnp.float32)

    out = pl.pallas_call(
        _kernel_body,
        grid=(B,),
        in_specs=[
            pl.BlockSpec((1, C, H * W), lambda b: (b, 0, 0)),
            pl.BlockSpec((C, C), lambda b: (0, 0)),
            pl.BlockSpec((C, 1), lambda b: (0, 0)),
            pl.BlockSpec(memory_space=pltpu.SMEM),
        ],
        out_specs=pl.BlockSpec((1, C, (H * W) // 4), lambda b: (b, 0, 0)),
        out_shape=jax.ShapeDtypeStruct((B, C, (H * W) // 4), jnp.float32),
        scratch_shapes=[
            pltpu.VMEM((_T // 2, _T // 2), jnp.float32),
            pltpu.VMEM((_T // 4, _T // 4), jnp.float32),
        ],
        compiler_params=pltpu.CompilerParams(
            dimension_semantics=("arbitrary",),
        ),
    )(xc, W_conv, b_conv.reshape(C, 1), fws)
    return out.reshape(B, C, H // 2, W // 2)


# speedup vs baseline: 3.7353x; 1.1413x over previous
import jax
import jax.numpy as jnp
from jax import lax
from jax.experimental import pallas as pl
from jax.experimental.pallas import tpu as pltpu

_C = 96
_T = 4096
_B = 8
_JC = 1024


def _fill_spatial(tab_ref, n, width, fw1, jc):
    ivec = lax.broadcasted_iota(jnp.int32, (n, 1), 0)
    a_row = (ivec // width).astype(jnp.float32)
    a_col = (ivec % width).astype(jnp.float32)
    for k in range(n // jc):
        off = k * jc
        jvec = lax.broadcasted_iota(jnp.int32, (1, jc), 1) + (n + off)
        b_row = (jvec // width).astype(jnp.float32)
        b_col = (jvec % width).astype(jnp.float32)
        dr = a_row - b_row
        dc = a_col - b_col
        dist = jnp.sqrt(dr * dr + dc * dc)
        spatial = 1.0 / (dist + 1e-6)
        tab_ref[:, off:off + jc] = fw1 * spatial


def _merge_pass(xb, n, fw0, jc, tab_ref):
    a = xb[:, :n]
    b = xb[:, n:]
    norm = jnp.sqrt(jnp.sum(xb * xb, axis=0, keepdims=True))
    a_n = (a / norm[:, :n]).astype(jnp.bfloat16)
    b_n = (b / norm[:, n:]).astype(jnp.bfloat16)

    best_val = jnp.full((n, 1), -jnp.inf, dtype=jnp.float32)
    best_idx = jnp.zeros((n, 1), dtype=jnp.int32)
    for k in range(n // jc):
        off = k * jc
        sim = lax.dot_general(
            a_n, b_n[:, off:off + jc],
            dimension_numbers=(((0,), (0,)), ((), ())),
            preferred_element_type=jnp.float32,
        )
        combined = fw0 * sim + tab_ref[:, off:off + jc]
        loc_max = jnp.max(combined, axis=1, keepdims=True)
        jj = lax.broadcasted_iota(jnp.int32, (n, jc), 1) + off
        loc_arg = jnp.min(
            jnp.where(combined == loc_max, jj, n), axis=1, keepdims=True
        )
        upd = loc_max > best_val
        best_val = jnp.where(upd, loc_max, best_val)
        best_idx = jnp.where(upd, loc_arg, best_idx)

    C = a.shape[0]
    a_hi = a.astype(jnp.bfloat16)
    a_hi = jnp.concatenate(
        [a_hi, jnp.ones((1, n), jnp.bfloat16)], axis=0)
    a_lo = (a - a_hi[:C].astype(jnp.float32)).astype(jnp.bfloat16)
    outs = []
    for k in range(n // jc):
        off = k * jc
        jj = lax.broadcasted_iota(jnp.int32, (n, jc), 1) + off
        onehot = (best_idx == jj).astype(jnp.bfloat16)
        dn = (((1,), (0,)), ((), ()))
        hi = lax.dot_general(a_hi, onehot, dimension_numbers=dn,
                             preferred_element_type=jnp.float32)
        lo = lax.dot_general(a_lo, onehot, dimension_numbers=dn,
                             preferred_element_type=jnp.float32)
        sums = b[:, off:off + jc] + hi[:C] + lo
        counts = 1.0 + hi[C:C + 1]
        outs.append(sums / counts)
    return jnp.concatenate(outs, axis=1)


def _kernel_body(x_ref, w_ref, b_ref, fw_ref, out_ref, tab1_ref, tab2_ref):
    @pl.when(pl.program_id(0) == 0)
    def _():
        _fill_spatial(tab1_ref, _T // 2, 64, fw_ref[0, 1], _JC)
        _fill_spatial(tab2_ref, _T // 4, 45, fw_ref[1, 1], _JC)

    xb = x_ref[0]
    m1 = _merge_pass(xb, _T // 2, fw_ref[0, 0], _JC, tab1_ref)
    m2 = _merge_pass(m1, _T // 4, fw_ref[1, 0], _JC, tab2_ref)
    out = lax.dot_general(
        w_ref[...], m2,
        dimension_numbers=(((1,), (0,)), ((), ())),
        preferred_element_type=jnp.float32,
        precision=lax.Precision.HIGHEST,
    ) + b_ref[...]
    out_ref[0] = out


@jax.jit
def kernel(x, W_conv, b_conv, w_fuse1, w_fuse2):
    B, C, H, W = x.shape
    xc = x.reshape(B, C, H * W)

    def fw(w):
        w = jnp.clip(w, 0.0, 6.0)
        return w / (jnp.sum(w) + 1e-8)

    fws = jnp.stack([fw(w_fuse1), fw(w_fuse2)]).astype(jnp.float32)

    out = pl.pallas_call(
        _kernel_body,
        grid=(B,),
        in_specs=[
            pl.BlockSpec((1, C, H * W), lambda b: (b, 0, 0)),
            pl.BlockSpec((C, C), lambda b: (0, 0)),
            pl.BlockSpec((C, 1), lambda b: (0, 0)),
            pl.BlockSpec(memory_space=pltpu.SMEM),
        ],
        out_specs=pl.BlockSpec((1, C, (H * W) // 4), lambda b: (b, 0, 0)),
        out_shape=jax.ShapeDtypeStruct((B, C, (H * W) // 4), jnp.float32),
        scratch_shapes=[
            pltpu.VMEM((_T // 2, _T // 2), jnp.float32),
            pltpu.VMEM((_T // 4, _T // 4), jnp.float32),
        ],
        compiler_params=pltpu.CompilerParams(
            dimension_semantics=("arbitrary",),
        ),
    )(xc, W_conv, b_conv.reshape(C, 1), fws)
    return out.reshape(B, C, H // 2, W // 2)
